# baseline (speedup 1.0000x reference)
import functools

import jax
import jax.numpy as jnp
from jax import lax
from jax.experimental import pallas as pl
from jax.experimental.pallas import tpu as pltpu
from jax.experimental.pallas import tpu_sc as plsc

VOCAB = 1000000
EMBED = 64
BATCH = 4096
SEQLEN = 200

NUM_CORES = 2
NUM_SUBCORES = 16
NW = NUM_CORES * NUM_SUBCORES
BPW = BATCH // NW
NCHUNK = 2
CHUNK = SEQLEN // NCHUNK
NVREG = EMBED // 16


def _bow_body(idx_hbm, table_hbm, out_hbm, idx_v, buf, out_v, sem):
    wid = lax.axis_index("s") * NUM_CORES + lax.axis_index("c")
    base = wid * BPW

    pltpu.sync_copy(idx_hbm.at[pl.ds(base, BPW)], idx_v)

    def row_body(b, _):
        h0 = pltpu.async_copy(
            table_hbm.at[idx_v.at[b, 0]], buf.at[pl.ds(0, CHUNK)], sem)
        h1 = pltpu.async_copy(
            table_hbm.at[idx_v.at[b, 1]], buf.at[pl.ds(CHUNK, CHUNK)], sem)
        h0.wait()
        h1.wait()

        accs = tuple(buf[0, pl.ds(16 * c, 16)] for c in range(NVREG))

        def red_body(r, accs):
            return tuple(
                jnp.maximum(a, buf[r, pl.ds(16 * c, 16)])
                for c, a in enumerate(accs))

        accs = lax.fori_loop(1, SEQLEN, red_body, accs, unroll=8)
        for c in range(NVREG):
            out_v[b, pl.ds(16 * c, 16)] = accs[c]
        return ()

    lax.fori_loop(0, BPW, row_body, ())

    pltpu.sync_copy(out_v, out_hbm.at[pl.ds(base, BPW)])


@functools.cache
def _bow():
    return functools.partial(
        pl.kernel,
        mesh=plsc.VectorSubcoreMesh(core_axis_name="c", subcore_axis_name="s"),
        out_type=jax.ShapeDtypeStruct((BATCH, EMBED), jnp.float32),
        scratch_types=[
            pltpu.VMEM((BPW, NCHUNK, CHUNK), jnp.int32),
            pltpu.VMEM((SEQLEN, EMBED), jnp.float32),
            pltpu.VMEM((BPW, EMBED), jnp.float32),
            pltpu.SemaphoreType.DMA,
        ],
        compiler_params=pltpu.CompilerParams(use_tc_tiling_on_sc=False),
    )(_bow_body)


@jax.jit
def kernel(input, table):
    idx = input.reshape(BATCH, NCHUNK, CHUNK)
    return _bow()(idx, table)
